# baseline (speedup 1.0000x reference)
import jax
import jax.numpy as jnp
from jax import lax
from jax.experimental import pallas as pl
from jax.experimental.pallas import tpu as pltpu
from jax.experimental.pallas import tpu_sc as plsc

H = W = 128
C = 128
N_ROIS = 1000
PH = PW = 7
SS = 2
P = N_ROIS * PH * PW
NC, NS, L = 2, 16, 16
NW = NC * NS
CHUNK = 4
PPW = 1536
CHUNKS = PPW // CHUNK
PAIRS = CHUNKS // 2
OUTP = NW * PPW


def _ci(v):
    return jnp.full((L,), v, jnp.int32)


def _cf(v):
    return jnp.full((L,), v, jnp.float32)


def _sc_body(table_hbm, rif_hbm, ppinit_hbm, out_hbm,
             rois_v, pp_v, idx_a, idx_b, w_a, w_b, rows_a, rows_b, out_v, sem):
    wid = lax.axis_index("s") * NC + lax.axis_index("c")
    pltpu.sync_copy(rif_hbm, rois_v)
    pltpu.sync_copy(ppinit_hbm.at[pl.ds(wid * L, L)], pp_v)

    def make_indices(idx_v, w_v):
        lane = lax.iota(jnp.int32, L)
        pp = pp_v[pl.ds(0, L)]
        pp_v[pl.ds(0, L)] = pp + _ci(CHUNK)
        ppc = jnp.minimum(pp, _ci(P - 1))
        s = jnp.bitwise_and(lane, _ci(3))
        sy = jnp.bitwise_and(lax.shift_right_logical(lane, _ci(1)), _ci(1))
        sx = jnp.bitwise_and(s, _ci(1))
        n0 = (ppc.astype(jnp.float32) * _cf(1.0 / 49.0)).astype(jnp.int32)
        r0 = ppc - n0 * _ci(49)
        n = n0 + _ci(1) + lax.shift_right_arithmetic(r0 - _ci(49), _ci(31))
        r = ppc - n * _ci(49)
        py0 = (r.astype(jnp.float32) * _cf(1.0 / 7.0)).astype(jnp.int32)
        q0 = r - py0 * _ci(7)
        py = py0 + _ci(1) + lax.shift_right_arithmetic(q0 - _ci(7), _ci(31))
        px = r - py * _ci(7)
        iy = (py + py + sy).astype(jnp.float32)
        ix = (px + px + sx).astype(jnp.float32)
        y0 = plsc.load_gather(rois_v, [n])
        x0 = plsc.load_gather(rois_v, [n + _ci(N_ROIS)])
        y2 = plsc.load_gather(rois_v, [n + _ci(2 * N_ROIS)])
        x2 = plsc.load_gather(rois_v, [n + _ci(3 * N_ROIS)])
        hs = (y2 - y0) / _cf(PH * SS)
        ws = (x2 - x0) / _cf(PW * SS)
        cy = iy * hs + hs * _cf(0.5) + y0
        cx = ix * ws + ws * _cf(0.5) + x0
        ty = cy.astype(jnp.int32)
        tx = cx.astype(jnp.int32)
        ly = cy - ty.astype(jnp.float32)
        lx = cx - tx.astype(jnp.float32)
        iu = jnp.minimum(jnp.maximum(ty, _ci(0)), _ci(H - 1))
        il = jnp.minimum(jnp.maximum(tx, _ci(0)), _ci(W - 1))
        incy = jnp.minimum((cy - iu.astype(jnp.float32)) * _cf(2.0 ** 30), _cf(1.0)).astype(jnp.int32)
        incx = jnp.minimum((cx - il.astype(jnp.float32)) * _cf(2.0 ** 30), _cf(1.0)).astype(jnp.int32)
        idn = jnp.minimum(iu + incy, _ci(H - 1))
        ir = jnp.minimum(il + incx, _ci(W - 1))
        omy = _cf(1.0) - ly
        omx = _cf(1.0) - lx
        iuw = iu * _ci(W)
        idw = idn * _ci(W)
        idx_v[pl.ds(0 * L, L)] = iuw + il
        idx_v[pl.ds(1 * L, L)] = idw + il
        idx_v[pl.ds(2 * L, L)] = iuw + ir
        idx_v[pl.ds(3 * L, L)] = idw + ir
        w_v[pl.ds(4 * L, L)] = omy * omx
        w_v[pl.ds(1 * L, L)] = ly * omx
        w_v[pl.ds(2 * L, L)] = omy * lx
        w_v[pl.ds(3 * L, L)] = ly * lx

    def fire(idx_v, rows_v):
        pltpu.async_copy(table_hbm.at[idx_v], rows_v, sem)

    def wait_for(idx_v, rows_v):
        pltpu.make_async_copy(table_hbm.at[idx_v], rows_v, sem).wait()

    def compute(rows_v, w_v, pbase):
        for pi in range(CHUNK):
            accs = [None] * (C // L)
            for sub in range(4):
                ln = pi * 4 + sub
                wk = [plsc.load_gather(w_v, [_ci((k if k else 4) * L + ln)])
                      for k in range(4)]
                for g in range(C // L):
                    sl = pl.ds(g * L, L)
                    v = (wk[0] * rows_v[0 * L + ln, sl]
                         + wk[1] * rows_v[1 * L + ln, sl]
                         + wk[2] * rows_v[2 * L + ln, sl]
                         + wk[3] * rows_v[3 * L + ln, sl])
                    accs[g] = v if sub == 0 else jnp.maximum(accs[g], v)
            for g in range(C // L):
                out_v[pi, pl.ds(g * L, L)] = accs[g]
        pltpu.sync_copy(out_v, out_hbm.at[pl.ds(pbase, CHUNK)])

    def chunk_body(t, carry):
        pbase = wid * PPW + t * CHUNK
        make_indices(idx_a, w_a)
        fire(idx_a, rows_a)
        wait_for(idx_a, rows_a)
        compute(rows_a, w_a, pbase)
        return carry

    lax.fori_loop(0, CHUNKS, chunk_body, 0)


def _roialign_sc(table, rif, ppinit):
    mesh = plsc.VectorSubcoreMesh(core_axis_name="c", subcore_axis_name="s")
    return pl.kernel(
        _sc_body,
        mesh=mesh,
        compiler_params=pltpu.CompilerParams(needs_layout_passes=False),
        out_type=jax.ShapeDtypeStruct((OUTP, C), jnp.float32),
        scratch_types=[
            pltpu.VMEM((4 * N_ROIS,), jnp.float32),
            pltpu.VMEM((L,), jnp.int32),
            pltpu.VMEM((4 * L,), jnp.int32),
            pltpu.VMEM((4 * L,), jnp.int32),
            pltpu.VMEM((5 * L,), jnp.float32),
            pltpu.VMEM((5 * L,), jnp.float32),
            pltpu.VMEM((4 * L, C), jnp.float32),
            pltpu.VMEM((4 * L, C), jnp.float32),
            pltpu.VMEM((CHUNK, C), jnp.float32),
            pltpu.SemaphoreType.DMA,
        ],
    )(table, rif, ppinit)


def kernel(features, rois, ratio):
    table = features[0].transpose(1, 2, 0).reshape(H * W, C)
    rif = (rois * ratio).astype(jnp.float32).T.reshape(4 * N_ROIS)
    ppinit = (jnp.arange(NW, dtype=jnp.int32)[:, None] * PPW
              + jnp.arange(L, dtype=jnp.int32)[None, :] // 4).reshape(-1)
    out = _roialign_sc(table, rif, ppinit)
    return out[:P].reshape(N_ROIS, PH, PW, C).transpose(0, 3, 1, 2)


# speedup vs baseline: 1.2846x; 1.2846x over previous
import jax
import jax.numpy as jnp
from jax import lax
from jax.experimental import pallas as pl
from jax.experimental.pallas import tpu as pltpu
from jax.experimental.pallas import tpu_sc as plsc

H = W = 128
C = 128
N_ROIS = 1000
PH = PW = 7
SS = 2
P = N_ROIS * PH * PW
NC, NS, L = 2, 16, 16
NW = NC * NS
CHUNK = 4
SUB = 2
STAGE = CHUNK * SUB
PPW = 1536
STAGES = PPW // STAGE
PAIRS = STAGES // 2
OUTP = NW * PPW
WSTRIDE = 5 * L


def _ci(v):
    return jnp.full((L,), v, jnp.int32)


def _cf(v):
    return jnp.full((L,), v, jnp.float32)


def _sc_body(table_hbm, rif_hbm, ppinit_hbm, out_hbm,
             rois_v, pp_v, idx_a, idx_b, w_a, w_b, rows_a, rows_b,
             out_a, out_b, sem, sem_oa, sem_ob):
    wid = lax.axis_index("s") * NC + lax.axis_index("c")
    pltpu.sync_copy(rif_hbm, rois_v)
    pltpu.sync_copy(ppinit_hbm.at[pl.ds(wid * L, L)], pp_v)

    def make_indices_block(idx_v, w_v, j):
        lane = lax.iota(jnp.int32, L)
        pp = pp_v[pl.ds(0, L)]
        pp_v[pl.ds(0, L)] = pp + _ci(CHUNK)
        ppc = jnp.minimum(pp, _ci(P - 1))
        s = jnp.bitwise_and(lane, _ci(3))
        sy = jnp.bitwise_and(lax.shift_right_logical(lane, _ci(1)), _ci(1))
        sx = jnp.bitwise_and(s, _ci(1))
        n0 = (ppc.astype(jnp.float32) * _cf(1.0 / 49.0)).astype(jnp.int32)
        r0 = ppc - n0 * _ci(49)
        n = n0 + _ci(1) + lax.shift_right_arithmetic(r0 - _ci(49), _ci(31))
        r = ppc - n * _ci(49)
        py0 = (r.astype(jnp.float32) * _cf(1.0 / 7.0)).astype(jnp.int32)
        q0 = r - py0 * _ci(7)
        py = py0 + _ci(1) + lax.shift_right_arithmetic(q0 - _ci(7), _ci(31))
        px = r - py * _ci(7)
        iy = (py + py + sy).astype(jnp.float32)
        ix = (px + px + sx).astype(jnp.float32)
        y0 = plsc.load_gather(rois_v, [n])
        x0 = plsc.load_gather(rois_v, [n + _ci(N_ROIS)])
        y2 = plsc.load_gather(rois_v, [n + _ci(2 * N_ROIS)])
        x2 = plsc.load_gather(rois_v, [n + _ci(3 * N_ROIS)])
        hs = (y2 - y0) / _cf(PH * SS)
        ws = (x2 - x0) / _cf(PW * SS)
        cy = iy * hs + hs * _cf(0.5) + y0
        cx = ix * ws + ws * _cf(0.5) + x0
        ty = cy.astype(jnp.int32)
        tx = cx.astype(jnp.int32)
        ly = cy - ty.astype(jnp.float32)
        lx = cx - tx.astype(jnp.float32)
        iu = jnp.minimum(jnp.maximum(ty, _ci(0)), _ci(H - 1))
        il = jnp.minimum(jnp.maximum(tx, _ci(0)), _ci(W - 1))
        incy = jnp.minimum((cy - iu.astype(jnp.float32)) * _cf(2.0 ** 30), _cf(1.0)).astype(jnp.int32)
        incx = jnp.minimum((cx - il.astype(jnp.float32)) * _cf(2.0 ** 30), _cf(1.0)).astype(jnp.int32)
        idn = jnp.minimum(iu + incy, _ci(H - 1))
        ir = jnp.minimum(il + incx, _ci(W - 1))
        omy = _cf(1.0) - ly
        omx = _cf(1.0) - lx
        iuw = iu * _ci(W)
        idw = idn * _ci(W)
        ib = j * 4 * L
        wb = j * WSTRIDE
        idx_v[pl.ds(ib + 0 * L, L)] = iuw + il
        idx_v[pl.ds(ib + 1 * L, L)] = idw + il
        idx_v[pl.ds(ib + 2 * L, L)] = iuw + ir
        idx_v[pl.ds(ib + 3 * L, L)] = idw + ir
        w_v[pl.ds(wb + 4 * L, L)] = omy * omx
        w_v[pl.ds(wb + 1 * L, L)] = ly * omx
        w_v[pl.ds(wb + 2 * L, L)] = omy * lx
        w_v[pl.ds(wb + 3 * L, L)] = ly * lx

    def make_indices(idx_v, w_v):
        for j in range(SUB):
            make_indices_block(idx_v, w_v, j)

    def fire(idx_v, rows_v):
        pltpu.async_copy(table_hbm.at[idx_v], rows_v, sem)

    def wait_for(idx_v, rows_v):
        pltpu.make_async_copy(table_hbm.at[idx_v], rows_v, sem).wait()

    def compute(rows_v, w_v, out_v, out_sem, pbase):
        pltpu.make_async_copy(out_v, out_hbm.at[pl.ds(pbase, STAGE)], out_sem).wait()
        for j in range(SUB):
            rb = j * 4 * L
            wb = j * WSTRIDE
            for pi in range(CHUNK):
                accs = [None] * (C // L)
                for sub in range(4):
                    ln = pi * 4 + sub
                    wk = [plsc.load_gather(w_v, [_ci(wb + (k if k else 4) * L + ln)])
                          for k in range(4)]
                    for g in range(C // L):
                        sl = pl.ds(g * L, L)
                        v = (wk[0] * rows_v[rb + 0 * L + ln, sl]
                             + wk[1] * rows_v[rb + 1 * L + ln, sl]
                             + wk[2] * rows_v[rb + 2 * L + ln, sl]
                             + wk[3] * rows_v[rb + 3 * L + ln, sl])
                        accs[g] = v if sub == 0 else jnp.maximum(accs[g], v)
                for g in range(C // L):
                    out_v[j * CHUNK + pi, pl.ds(g * L, L)] = accs[g]
        pltpu.async_copy(out_v, out_hbm.at[pl.ds(pbase, STAGE)], out_sem)

    DUMMY = OUTP - STAGE
    make_indices(idx_a, w_a)
    fire(idx_a, rows_a)
    pltpu.async_copy(out_a, out_hbm.at[pl.ds(DUMMY, STAGE)], sem_oa)
    pltpu.async_copy(out_b, out_hbm.at[pl.ds(DUMMY, STAGE)], sem_ob)

    def pair_body(i, carry):
        base0 = wid * PPW + i * (2 * STAGE)
        make_indices(idx_b, w_b)
        wait_for(idx_a, rows_a)
        fire(idx_b, rows_b)
        compute(rows_a, w_a, out_a, sem_oa, base0)
        make_indices(idx_a, w_a)
        wait_for(idx_b, rows_b)
        fire(idx_a, rows_a)
        compute(rows_b, w_b, out_b, sem_ob, base0 + STAGE)
        return carry

    lax.fori_loop(0, PAIRS - 1, pair_body, 0)
    tail = wid * PPW + (STAGES - 2) * STAGE
    make_indices(idx_b, w_b)
    wait_for(idx_a, rows_a)
    fire(idx_b, rows_b)
    compute(rows_a, w_a, out_a, sem_oa, tail)
    wait_for(idx_b, rows_b)
    compute(rows_b, w_b, out_b, sem_ob, tail + STAGE)
    pltpu.make_async_copy(out_a, out_hbm.at[pl.ds(tail, STAGE)], sem_oa).wait()
    pltpu.make_async_copy(out_b, out_hbm.at[pl.ds(tail + STAGE, STAGE)], sem_ob).wait()


def _roialign_sc(table, rif, ppinit):
    mesh = plsc.VectorSubcoreMesh(core_axis_name="c", subcore_axis_name="s")
    return pl.kernel(
        _sc_body,
        mesh=mesh,
        compiler_params=pltpu.CompilerParams(needs_layout_passes=False),
        out_type=jax.ShapeDtypeStruct((OUTP, C), jnp.float32),
        scratch_types=[
            pltpu.VMEM((4 * N_ROIS,), jnp.float32),
            pltpu.VMEM((L,), jnp.int32),
            pltpu.VMEM((SUB * 4 * L,), jnp.int32),
            pltpu.VMEM((SUB * 4 * L,), jnp.int32),
            pltpu.VMEM((SUB * WSTRIDE,), jnp.float32),
            pltpu.VMEM((SUB * WSTRIDE,), jnp.float32),
            pltpu.VMEM((SUB * 4 * L, C), jnp.float32),
            pltpu.VMEM((SUB * 4 * L, C), jnp.float32),
            pltpu.VMEM((STAGE, C), jnp.float32),
            pltpu.VMEM((STAGE, C), jnp.float32),
            pltpu.SemaphoreType.DMA,
            pltpu.SemaphoreType.DMA,
            pltpu.SemaphoreType.DMA,
        ],
    )(table, rif, ppinit)


def kernel(features, rois, ratio):
    table = features[0].transpose(1, 2, 0).reshape(H * W, C)
    rif = (rois * ratio).astype(jnp.float32).T.reshape(4 * N_ROIS)
    ppinit = (jnp.arange(NW, dtype=jnp.int32)[:, None] * PPW
              + jnp.arange(L, dtype=jnp.int32)[None, :] // 4).reshape(-1)
    out = _roialign_sc(table, rif, ppinit)
    return out[:P].reshape(N_ROIS, PH, PW, C).transpose(0, 3, 1, 2)


# speedup vs baseline: 1.3940x; 1.0852x over previous
import jax
import jax.numpy as jnp
from jax import lax
from jax.experimental import pallas as pl
from jax.experimental.pallas import tpu as pltpu
from jax.experimental.pallas import tpu_sc as plsc

H = W = 128
C = 128
N_ROIS = 1000
PH = PW = 7
SS = 2
P = N_ROIS * PH * PW
NC, NS, L = 2, 16, 16
NW = NC * NS
CHUNK = 4
PPW = 1536
CHUNKS = PPW // CHUNK
TRIPLES = CHUNKS // 3 - 1
OUTP = NW * PPW


def _ci(v):
    return jnp.full((L,), v, jnp.int32)


def _cf(v):
    return jnp.full((L,), v, jnp.float32)


def _sc_body(table_hbm, rif_hbm, ppinit_hbm, out_hbm,
             rois_v, pp_v, idx_a, idx_b, idx_c, w_a, w_b, w_c,
             rows_a, rows_b, rows_c, out_v, sem_a, sem_b, sem_c):
    wid = lax.axis_index("s") * NC + lax.axis_index("c")
    pltpu.sync_copy(rif_hbm, rois_v)
    pltpu.sync_copy(ppinit_hbm.at[pl.ds(wid * L, L)], pp_v)

    def make_indices(idx_v, w_v):
        lane = lax.iota(jnp.int32, L)
        pp = pp_v[pl.ds(0, L)]
        pp_v[pl.ds(0, L)] = pp + _ci(CHUNK)
        ppc = jnp.minimum(pp, _ci(P - 1))
        s = jnp.bitwise_and(lane, _ci(3))
        sy = jnp.bitwise_and(lax.shift_right_logical(lane, _ci(1)), _ci(1))
        sx = jnp.bitwise_and(s, _ci(1))
        n0 = (ppc.astype(jnp.float32) * _cf(1.0 / 49.0)).astype(jnp.int32)
        r0 = ppc - n0 * _ci(49)
        n = n0 + _ci(1) + lax.shift_right_arithmetic(r0 - _ci(49), _ci(31))
        r = ppc - n * _ci(49)
        py0 = (r.astype(jnp.float32) * _cf(1.0 / 7.0)).astype(jnp.int32)
        q0 = r - py0 * _ci(7)
        py = py0 + _ci(1) + lax.shift_right_arithmetic(q0 - _ci(7), _ci(31))
        px = r - py * _ci(7)
        iy = (py + py + sy).astype(jnp.float32)
        ix = (px + px + sx).astype(jnp.float32)
        y0 = plsc.load_gather(rois_v, [n])
        x0 = plsc.load_gather(rois_v, [n + _ci(N_ROIS)])
        y2 = plsc.load_gather(rois_v, [n + _ci(2 * N_ROIS)])
        x2 = plsc.load_gather(rois_v, [n + _ci(3 * N_ROIS)])
        hs = (y2 - y0) / _cf(PH * SS)
        ws = (x2 - x0) / _cf(PW * SS)
        cy = iy * hs + hs * _cf(0.5) + y0
        cx = ix * ws + ws * _cf(0.5) + x0
        ty = cy.astype(jnp.int32)
        tx = cx.astype(jnp.int32)
        ly = cy - ty.astype(jnp.float32)
        lx = cx - tx.astype(jnp.float32)
        iu = jnp.minimum(jnp.maximum(ty, _ci(0)), _ci(H - 1))
        il = jnp.minimum(jnp.maximum(tx, _ci(0)), _ci(W - 1))
        incy = jnp.minimum((cy - iu.astype(jnp.float32)) * _cf(2.0 ** 30), _cf(1.0)).astype(jnp.int32)
        incx = jnp.minimum((cx - il.astype(jnp.float32)) * _cf(2.0 ** 30), _cf(1.0)).astype(jnp.int32)
        idn = jnp.minimum(iu + incy, _ci(H - 1))
        ir = jnp.minimum(il + incx, _ci(W - 1))
        omy = _cf(1.0) - ly
        omx = _cf(1.0) - lx
        iuw = iu * _ci(W)
        idw = idn * _ci(W)
        idx_v[pl.ds(0 * L, L)] = iuw + il
        idx_v[pl.ds(1 * L, L)] = idw + il
        idx_v[pl.ds(2 * L, L)] = iuw + ir
        idx_v[pl.ds(3 * L, L)] = idw + ir
        w_v[pl.ds(4 * L, L)] = omy * omx
        w_v[pl.ds(1 * L, L)] = ly * omx
        w_v[pl.ds(2 * L, L)] = omy * lx
        w_v[pl.ds(3 * L, L)] = ly * lx

    def fire(idx_v, rows_v, sem):
        pltpu.async_copy(table_hbm.at[idx_v], rows_v, sem)

    def wait_for(idx_v, rows_v, sem):
        pltpu.make_async_copy(table_hbm.at[idx_v], rows_v, sem).wait()

    def compute(rows_v, w_v, pbase):
        for pi in range(CHUNK):
            accs = [None] * (C // L)
            for sub in range(4):
                ln = pi * 4 + sub
                wk = [plsc.load_gather(w_v, [_ci((k if k else 4) * L + ln)])
                      for k in range(4)]
                for g in range(C // L):
                    sl = pl.ds(g * L, L)
                    v = (wk[0] * rows_v[0 * L + ln, sl]
                         + wk[1] * rows_v[1 * L + ln, sl]
                         + wk[2] * rows_v[2 * L + ln, sl]
                         + wk[3] * rows_v[3 * L + ln, sl])
                    accs[g] = v if sub == 0 else jnp.maximum(accs[g], v)
            for g in range(C // L):
                out_v[pi, pl.ds(g * L, L)] = accs[g]
        pltpu.sync_copy(out_v, out_hbm.at[pl.ds(pbase, CHUNK)])

    bufs = ((idx_a, rows_a, w_a, sem_a),
            (idx_b, rows_b, w_b, sem_b),
            (idx_c, rows_c, w_c, sem_c))

    make_indices(idx_a, w_a)
    fire(idx_a, rows_a, sem_a)
    make_indices(idx_b, w_b)
    fire(idx_b, rows_b, sem_b)

    def ring_body(i, carry):
        t0 = i * 3
        for slot in range(3):
            nidx, nrows, nw, nsem = bufs[(slot + 2) % 3]
            cidx, crows, cw, csem = bufs[slot]
            make_indices(nidx, nw)
            fire(nidx, nrows, nsem)
            wait_for(cidx, crows, csem)
            compute(crows, cw, wid * PPW + (t0 + slot) * CHUNK)
        return carry

    lax.fori_loop(0, TRIPLES, ring_body, 0)
    tail = TRIPLES * 3
    make_indices(idx_c, w_c)
    fire(idx_c, rows_c, sem_c)
    wait_for(idx_a, rows_a, sem_a)
    compute(rows_a, w_a, wid * PPW + tail * CHUNK)
    wait_for(idx_b, rows_b, sem_b)
    compute(rows_b, w_b, wid * PPW + (tail + 1) * CHUNK)
    wait_for(idx_c, rows_c, sem_c)
    compute(rows_c, w_c, wid * PPW + (tail + 2) * CHUNK)


def _roialign_sc(table, rif, ppinit):
    mesh = plsc.VectorSubcoreMesh(core_axis_name="c", subcore_axis_name="s")
    return pl.kernel(
        _sc_body,
        mesh=mesh,
        compiler_params=pltpu.CompilerParams(needs_layout_passes=False),
        out_type=jax.ShapeDtypeStruct((OUTP, C), jnp.float32),
        scratch_types=[
            pltpu.VMEM((4 * N_ROIS,), jnp.float32),
            pltpu.VMEM((L,), jnp.int32),
            pltpu.VMEM((4 * L,), jnp.int32),
            pltpu.VMEM((4 * L,), jnp.int32),
            pltpu.VMEM((4 * L,), jnp.int32),
            pltpu.VMEM((5 * L,), jnp.float32),
            pltpu.VMEM((5 * L,), jnp.float32),
            pltpu.VMEM((5 * L,), jnp.float32),
            pltpu.VMEM((4 * L, C), jnp.float32),
            pltpu.VMEM((4 * L, C), jnp.float32),
            pltpu.VMEM((4 * L, C), jnp.float32),
            pltpu.VMEM((CHUNK, C), jnp.float32),
            pltpu.SemaphoreType.DMA,
            pltpu.SemaphoreType.DMA,
            pltpu.SemaphoreType.DMA,
        ],
    )(table, rif, ppinit)


def kernel(features, rois, ratio):
    table = features[0].transpose(1, 2, 0).reshape(H * W, C)
    rif = (rois * ratio).astype(jnp.float32).T.reshape(4 * N_ROIS)
    ppinit = (jnp.arange(NW, dtype=jnp.int32)[:, None] * PPW
              + jnp.arange(L, dtype=jnp.int32)[None, :] // 4).reshape(-1)
    out = _roialign_sc(table, rif, ppinit)
    return out[:P].reshape(N_ROIS, PH, PW, C).transpose(0, 3, 1, 2)
